# baseline (speedup 1.0000x reference)
import jax
import jax.numpy as jnp
from jax.experimental import pallas as pl
from jax.experimental.pallas import tpu as pltpu

_B, _N, _D, _P = 4, 3136, 64, 8192
_H = 56
_QT = 448
_DA = 72


def _min_dist_kernel(qa_ref, ct_ref, out_ref, cta_ref):
    @pl.when(pl.program_id(0) == 0)
    def _():
        ct = ct_ref[...]
        cnorm = jnp.sum(ct * ct, axis=0, keepdims=True)
        pad = jnp.zeros((_DA - _D - 1, _P), jnp.float32)
        cta_ref[...] = jnp.concatenate([-2.0 * ct, cnorm, pad], axis=0)

    qa = qa_ref[...]
    dist = jnp.dot(qa, cta_ref[...],
                   preferred_element_type=jnp.float32)
    m = jnp.min(dist, axis=1, keepdims=True)
    qnorm = jnp.sum(qa[:, :_D] * qa[:, :_D], axis=1, keepdims=True)
    out_ref[...] = jnp.sqrt(m + qnorm)


@jax.jit
def kernel(embeds, centroids, r):
    del r
    q = embeds.reshape(_B * _N, _D)
    qa = jnp.concatenate(
        [q, jnp.ones((_B * _N, 1), jnp.float32),
         jnp.zeros((_B * _N, _DA - _D - 1), jnp.float32)], axis=1)
    ct = centroids.T
    out = pl.pallas_call(
        _min_dist_kernel,
        grid=(_B * _N // _QT,),
        in_specs=[
            pl.BlockSpec((_QT, _DA), lambda i: (i, 0)),
            pl.BlockSpec((_D, _P), lambda i: (0, 0)),
        ],
        out_specs=pl.BlockSpec((_QT, 1), lambda i: (i, 0)),
        out_shape=jax.ShapeDtypeStruct((_B * _N, 1), jnp.float32),
        scratch_shapes=[pltpu.VMEM((_DA, _P), jnp.float32)],
        compiler_params=pltpu.CompilerParams(
            dimension_semantics=("arbitrary",)),
    )(qa, ct)
    score = jnp.transpose(out.reshape(_B, _H, _H, 1), (0, 3, 1, 2))
    return (jnp.float32(0.0), score)


# speedup vs baseline: 1.0664x; 1.0664x over previous
import jax
import jax.numpy as jnp
from jax.experimental import pallas as pl
from jax.experimental.pallas import tpu as pltpu

_B, _N, _D, _P = 4, 3136, 64, 8192
_H = 56
_QT = 448


def _min_dist_kernel(q_ref, ct_ref, out_ref):
    q = q_ref[...]
    ct = ct_ref[...]
    qs = -2.0 * q
    dots = jnp.dot(qs, ct, preferred_element_type=jnp.float32)
    ctf = ct.astype(jnp.float32)
    cnorm = jnp.sum(ctf * ctf, axis=0)
    m = jnp.min(cnorm[None, :] + dots, axis=1, keepdims=True)
    qf = q.astype(jnp.float32)
    qnorm = jnp.sum(qf * qf, axis=1, keepdims=True)
    out_ref[...] = jnp.sqrt(m + qnorm)


@jax.jit
def kernel(embeds, centroids, r):
    del r
    q = embeds.reshape(_B * _N, _D).astype(jnp.bfloat16)
    ct = centroids.T.astype(jnp.bfloat16)
    out = pl.pallas_call(
        _min_dist_kernel,
        grid=(_B * _N // _QT,),
        in_specs=[
            pl.BlockSpec((_QT, _D), lambda i: (i, 0)),
            pl.BlockSpec((_D, _P), lambda i: (0, 0)),
        ],
        out_specs=pl.BlockSpec((_QT, 1), lambda i: (i, 0)),
        out_shape=jax.ShapeDtypeStruct((_B * _N, 1), jnp.float32),
        compiler_params=pltpu.CompilerParams(
            dimension_semantics=("parallel",)),
    )(q, ct)
    score = jnp.transpose(out.reshape(_B, _H, _H, 1), (0, 3, 1, 2))
    return (jnp.float32(0.0), score)


# speedup vs baseline: 1.1289x; 1.0586x over previous
import jax
import jax.numpy as jnp
from jax.experimental import pallas as pl
from jax.experimental.pallas import tpu as pltpu

_B, _N, _D, _P = 4, 3136, 64, 8192
_H = 56
_QT = 896


def _min_dist_kernel(q_ref, ct_ref, out_ref):
    q = q_ref[...]
    ct = ct_ref[...]
    qs = -2.0 * q
    dots = jnp.dot(qs, ct, preferred_element_type=jnp.float32)
    ctf = ct.astype(jnp.float32)
    cnorm = jnp.sum(ctf * ctf, axis=0)
    m = jnp.min(cnorm[None, :] + dots, axis=1, keepdims=True)
    qf = q.astype(jnp.float32)
    qnorm = jnp.sum(qf * qf, axis=1, keepdims=True)
    out_ref[...] = jnp.sqrt(m + qnorm)


@jax.jit
def kernel(embeds, centroids, r):
    del r
    q = embeds.reshape(_B * _N, _D).astype(jnp.bfloat16)
    ct = centroids.T.astype(jnp.bfloat16)
    out = pl.pallas_call(
        _min_dist_kernel,
        grid=(_B * _N // _QT,),
        in_specs=[
            pl.BlockSpec((_QT, _D), lambda i: (i, 0)),
            pl.BlockSpec((_D, _P), lambda i: (0, 0)),
        ],
        out_specs=pl.BlockSpec((_QT, 1), lambda i: (i, 0)),
        out_shape=jax.ShapeDtypeStruct((_B * _N, 1), jnp.float32),
        compiler_params=pltpu.CompilerParams(
            dimension_semantics=("parallel",)),
    )(q, ct)
    score = jnp.transpose(out.reshape(_B, _H, _H, 1), (0, 3, 1, 2))
    return (jnp.float32(0.0), score)
